# baseline (speedup 1.0000x reference)
import functools

import jax
import jax.numpy as jnp
from jax import lax
from jax.experimental import pallas as pl
from jax.experimental.pallas import tpu as pltpu
from jax.experimental.pallas import tpu_sc as plsc


def _argmin_body(x_ref, cb_ref, idx_ref):
    x = x_ref[...]
    cb = cb_ref[...]
    s = lax.dot_general(
        x, cb, (((1,), (1,)), ((), ())),
        preferred_element_type=jnp.float32,
    )
    x_sq = jnp.sum(x * x, axis=1, keepdims=True)
    c_sq = jnp.sum(cb * cb, axis=1)[None, :]
    d2 = x_sq - 2.0 * s + c_sq
    dist = jnp.sqrt(jnp.maximum(d2, 0.0))
    k = dist.shape[1]
    m = jnp.min(dist, axis=1, keepdims=True)
    iota = lax.broadcasted_iota(jnp.int32, dist.shape, 1)
    idx = jnp.min(jnp.where(dist == m, iota, k), axis=1)
    idx_ref[0, 0, :] = idx


def _tc_argmin(flat_x, codebook, block_n):
    n, d = flat_x.shape
    k = codebook.shape[0]
    nb = n // block_n
    out = pl.pallas_call(
        _argmin_body,
        grid=(nb,),
        in_specs=[
            pl.BlockSpec((block_n, d), lambda i: (i, 0)),
            pl.BlockSpec((k, d), lambda i: (0, 0)),
        ],
        out_specs=pl.BlockSpec((1, 1, block_n), lambda i: (i, 0, 0)),
        out_shape=jax.ShapeDtypeStruct((nb, 1, block_n), jnp.int32),
    )(flat_x, codebook)
    return out.reshape(n)


def _sc_info():
    try:
        info = plsc.get_sparse_core_info()
        return info.num_cores, info.num_subcores
    except Exception:
        return 2, 16


def _make_sc_gather(n, d, chunk):
    nc, ns = _sc_info()
    nw = nc * ns
    b_per_w = n // nw
    assert b_per_w % chunk == 0 and chunk % 8 == 0
    n_chunks = b_per_w // chunk
    mesh = plsc.VectorSubcoreMesh(core_axis_name="c", subcore_axis_name="s")

    @functools.partial(
        pl.kernel,
        mesh=mesh,
        out_type=jax.ShapeDtypeStruct((n, d), jnp.float32),
        scratch_types=[
            pltpu.VMEM((b_per_w,), jnp.int32),
            pltpu.VMEM((chunk, d), jnp.float32),
            pltpu.VMEM((chunk, d), jnp.float32),
            pltpu.SemaphoreType.DMA,
            pltpu.SemaphoreType.DMA,
            pltpu.SemaphoreType.DMA,
            pltpu.SemaphoreType.DMA,
        ],
    )
    def gather_kernel(table_hbm, idx_hbm, out_hbm,
                      idx_v, rows0, rows1, gsem0, gsem1, ssem0, ssem1):
        wid = lax.axis_index("s") * nc + lax.axis_index("c")
        base = wid * b_per_w
        rows = (rows0, rows1)
        gsem = (gsem0, gsem1)
        ssem = (ssem0, ssem1)
        pltpu.sync_copy(idx_hbm.at[pl.ds(base, b_per_w)], idx_v)
        stores = [None, None]
        gath = pltpu.async_copy(
            table_hbm.at[idx_v.at[pl.ds(0, chunk)]], rows[0], gsem[0])
        for c in range(n_chunks):
            b = c & 1
            gath.wait()
            if c + 1 < n_chunks:
                b2 = (c + 1) & 1
                if stores[b2] is not None:
                    stores[b2].wait()
                gath = pltpu.async_copy(
                    table_hbm.at[idx_v.at[pl.ds((c + 1) * chunk, chunk)]],
                    rows[b2], gsem[b2])
            stores[b] = pltpu.async_copy(
                rows[b], out_hbm.at[pl.ds(base + c * chunk, chunk)], ssem[b])
        for st in stores:
            if st is not None:
                st.wait()

    return gather_kernel


def kernel(x, codebook):
    d = x.shape[-1]
    flat_x = x.reshape(-1, d)
    n = flat_x.shape[0]
    parts = 2
    np_ = n // parts
    gather = _make_sc_gather(np_, d, chunk=128)
    outs = []
    for p in range(parts):
        xs = lax.slice(flat_x, (p * np_, 0), ((p + 1) * np_, d))
        idx = _tc_argmin(xs, codebook, block_n=2048)
        outs.append(gather(codebook, idx))
    return jnp.concatenate(outs, axis=0).reshape(x.shape)


# speedup vs baseline: 1.2283x; 1.2283x over previous
import functools

import jax
import jax.numpy as jnp
from jax import lax
from jax.experimental import pallas as pl
from jax.experimental.pallas import tpu as pltpu
from jax.experimental.pallas import tpu_sc as plsc


def _argmin_body(x_ref, cb_ref, idx_ref):
    x = x_ref[...]
    cb = cb_ref[...]
    s = lax.dot_general(
        x, cb, (((1,), (1,)), ((), ())),
        preferred_element_type=jnp.float32,
    )
    x_sq = jnp.sum(x * x, axis=1, keepdims=True)
    c_sq = jnp.sum(cb * cb, axis=1)[None, :]
    d2 = x_sq - 2.0 * s + c_sq
    dist = jnp.sqrt(jnp.maximum(d2, 0.0))
    k = dist.shape[1]
    m = jnp.min(dist, axis=1, keepdims=True)
    iota = lax.broadcasted_iota(jnp.int32, dist.shape, 1)
    idx = jnp.min(jnp.where(dist == m, iota, k), axis=1)
    idx_ref[0, 0, :] = idx


def _tc_argmin(flat_x, codebook, block_n, block_off, nb):
    d = flat_x.shape[1]
    k = codebook.shape[0]
    out = pl.pallas_call(
        _argmin_body,
        grid=(nb,),
        in_specs=[
            pl.BlockSpec((block_n, d), lambda i: (i + block_off, 0)),
            pl.BlockSpec((k, d), lambda i: (0, 0)),
        ],
        out_specs=pl.BlockSpec((1, 1, block_n), lambda i: (i, 0, 0)),
        out_shape=jax.ShapeDtypeStruct((nb, 1, block_n), jnp.int32),
    )(flat_x, codebook)
    return out.reshape(nb * block_n)


def _sc_info():
    try:
        info = plsc.get_sparse_core_info()
        return info.num_cores, info.num_subcores
    except Exception:
        return 2, 16


def _make_sc_gather(n_part, d, part_off, chunk):
    nc, ns = _sc_info()
    nw = nc * ns
    b_per_w = n_part // nw
    assert b_per_w % chunk == 0 and chunk % 8 == 0
    n_chunks = b_per_w // chunk
    mesh = plsc.VectorSubcoreMesh(core_axis_name="c", subcore_axis_name="s")

    @functools.partial(
        pl.kernel,
        mesh=mesh,
        scratch_types=[
            pltpu.VMEM((b_per_w,), jnp.int32),
            pltpu.VMEM((chunk, d), jnp.float32),
            pltpu.VMEM((chunk, d), jnp.float32),
            pltpu.SemaphoreType.DMA,
            pltpu.SemaphoreType.DMA,
            pltpu.SemaphoreType.DMA,
            pltpu.SemaphoreType.DMA,
        ],
    )
    def gather_kernel(table_hbm, idx_hbm, out_hbm,
                      idx_v, rows0, rows1, gsem0, gsem1, ssem0, ssem1):
        wid = lax.axis_index("s") * nc + lax.axis_index("c")
        base = wid * b_per_w
        rows = (rows0, rows1)
        gsem = (gsem0, gsem1)
        ssem = (ssem0, ssem1)
        pltpu.sync_copy(idx_hbm.at[pl.ds(base, b_per_w)], idx_v)
        stores = [None, None]
        gath = pltpu.async_copy(
            table_hbm.at[idx_v.at[pl.ds(0, chunk)]], rows[0], gsem[0])
        for c in range(n_chunks):
            b = c & 1
            gath.wait()
            if c + 1 < n_chunks:
                b2 = (c + 1) & 1
                if stores[b2] is not None:
                    stores[b2].wait()
                gath = pltpu.async_copy(
                    table_hbm.at[idx_v.at[pl.ds((c + 1) * chunk, chunk)]],
                    rows[b2], gsem[b2])
            stores[b] = pltpu.async_copy(
                rows[b],
                out_hbm.at[pl.ds(part_off + base + c * chunk, chunk)],
                ssem[b])
        for st in stores:
            if st is not None:
                st.wait()

    return gather_kernel


def kernel(x, codebook):
    d = x.shape[-1]
    flat_x = x.reshape(-1, d)
    n = flat_x.shape[0]
    parts = 2
    block_n = 2048
    np_ = n // parts
    nb = np_ // block_n
    out_ref = jax.new_ref(jnp.zeros((n, d), jnp.float32))
    for p in range(parts):
        idx = _tc_argmin(flat_x, codebook, block_n, p * nb, nb)
        _make_sc_gather(np_, d, p * np_, chunk=128)(codebook, idx, out_ref)
    return out_ref[...].reshape(x.shape)


# speedup vs baseline: 1.2761x; 1.0389x over previous
import functools

import jax
import jax.numpy as jnp
from jax import lax
from jax.experimental import pallas as pl
from jax.experimental.pallas import tpu as pltpu
from jax.experimental.pallas import tpu_sc as plsc


def _argmin_body(x_ref, cb_ref, idx_ref):
    x = x_ref[...]
    cb = cb_ref[...]
    s = lax.dot_general(
        x, cb, (((1,), (1,)), ((), ())),
        preferred_element_type=jnp.float32,
    )
    x_sq = jnp.sum(x * x, axis=1, keepdims=True)
    c_sq = jnp.sum(cb * cb, axis=1)[None, :]
    d2 = x_sq - 2.0 * s + c_sq
    dist = jnp.sqrt(jnp.maximum(d2, 0.0))
    k = dist.shape[1]
    m = jnp.min(dist, axis=1, keepdims=True)
    iota = lax.broadcasted_iota(jnp.int32, dist.shape, 1)
    idx = jnp.min(jnp.where(dist == m, iota, k), axis=1)
    idx_ref[0, 0, :] = idx


def _tc_argmin(flat_x, codebook, block_n, block_off, nb):
    d = flat_x.shape[1]
    k = codebook.shape[0]
    out = pl.pallas_call(
        _argmin_body,
        grid=(nb,),
        in_specs=[
            pl.BlockSpec((block_n, d), lambda i: (i + block_off, 0)),
            pl.BlockSpec((k, d), lambda i: (0, 0)),
        ],
        out_specs=pl.BlockSpec((1, 1, block_n), lambda i: (i, 0, 0)),
        out_shape=jax.ShapeDtypeStruct((nb, 1, block_n), jnp.int32),
    )(flat_x, codebook)
    return out.reshape(nb * block_n)


def _sc_info():
    try:
        info = plsc.get_sparse_core_info()
        return info.num_cores, info.num_subcores
    except Exception:
        return 2, 16


def _make_sc_gather(n_part, d, part_off, chunk):
    nc, ns = _sc_info()
    nw = nc * ns
    b_per_w = n_part // nw
    assert b_per_w % chunk == 0 and chunk % 8 == 0
    n_chunks = b_per_w // chunk
    mesh = plsc.VectorSubcoreMesh(core_axis_name="c", subcore_axis_name="s")

    @functools.partial(
        pl.kernel,
        mesh=mesh,
        scratch_types=[
            pltpu.VMEM((b_per_w,), jnp.int32),
            pltpu.VMEM((chunk, d), jnp.float32),
            pltpu.VMEM((chunk, d), jnp.float32),
            pltpu.SemaphoreType.DMA,
            pltpu.SemaphoreType.DMA,
            pltpu.SemaphoreType.DMA,
            pltpu.SemaphoreType.DMA,
        ],
    )
    def gather_kernel(table_hbm, idx_hbm, out_hbm,
                      idx_v, rows0, rows1, gsem0, gsem1, ssem0, ssem1):
        wid = lax.axis_index("s") * nc + lax.axis_index("c")
        base = wid * b_per_w
        rows = (rows0, rows1)
        gsem = (gsem0, gsem1)
        ssem = (ssem0, ssem1)
        pltpu.sync_copy(idx_hbm.at[pl.ds(base, b_per_w)], idx_v)
        stores = [None, None]
        gath = pltpu.async_copy(
            table_hbm.at[idx_v.at[pl.ds(0, chunk)]], rows[0], gsem[0])
        for c in range(n_chunks):
            b = c & 1
            gath.wait()
            if c + 1 < n_chunks:
                b2 = (c + 1) & 1
                if stores[b2] is not None:
                    stores[b2].wait()
                gath = pltpu.async_copy(
                    table_hbm.at[idx_v.at[pl.ds((c + 1) * chunk, chunk)]],
                    rows[b2], gsem[b2])
            stores[b] = pltpu.async_copy(
                rows[b],
                out_hbm.at[pl.ds(part_off + base + c * chunk, chunk)],
                ssem[b])
        for st in stores:
            if st is not None:
                st.wait()

    return gather_kernel


def _alloc_body(o_ref):
    o_ref[...] = jnp.zeros_like(o_ref)


def _alloc_uninit(n, d):
    return pl.pallas_call(
        _alloc_body,
        grid=(1,),
        out_specs=pl.BlockSpec((8, 128), lambda i: (0, 0)),
        out_shape=jax.ShapeDtypeStruct((n, d), jnp.float32),
    )()


def kernel(x, codebook):
    d = x.shape[-1]
    flat_x = x.reshape(-1, d)
    n = flat_x.shape[0]
    parts = 2
    block_n = 2048
    np_ = n // parts
    nb = np_ // block_n
    out_ref = jax.new_ref(_alloc_uninit(n, d))
    for p in range(parts):
        idx = _tc_argmin(flat_x, codebook, block_n, p * nb, nb)
        _make_sc_gather(np_, d, p * np_, chunk=64)(codebook, idx, out_ref)
    return out_ref[...].reshape(x.shape)


# speedup vs baseline: 1.3231x; 1.0368x over previous
import functools

import jax
import jax.numpy as jnp
from jax import lax
from jax.experimental import pallas as pl
from jax.experimental.pallas import tpu as pltpu
from jax.experimental.pallas import tpu_sc as plsc


def _argmin_body(x_ref, cb_ref, idx_ref):
    x = x_ref[...]
    cb = cb_ref[...]
    s = lax.dot_general(
        x, cb, (((1,), (1,)), ((), ())),
        preferred_element_type=jnp.float32,
    )
    x_sq = jnp.sum(x * x, axis=1, keepdims=True)
    c_sq = jnp.sum(cb * cb, axis=1)[None, :]
    d2 = x_sq - 2.0 * s + c_sq
    dist = jnp.sqrt(jnp.maximum(d2, 0.0))
    k = dist.shape[1]
    m = jnp.min(dist, axis=1, keepdims=True)
    iota = lax.broadcasted_iota(jnp.int32, dist.shape, 1)
    idx = jnp.min(jnp.where(dist == m, iota, k), axis=1)
    idx_ref[0, 0, :] = idx


def _tc_argmin(flat_x, codebook, block_n, block_off, nb):
    d = flat_x.shape[1]
    k = codebook.shape[0]
    out = pl.pallas_call(
        _argmin_body,
        grid=(nb,),
        in_specs=[
            pl.BlockSpec((block_n, d), lambda i: (i + block_off, 0)),
            pl.BlockSpec((k, d), lambda i: (0, 0)),
        ],
        out_specs=pl.BlockSpec((1, 1, block_n), lambda i: (i, 0, 0)),
        out_shape=jax.ShapeDtypeStruct((nb, 1, block_n), jnp.int32),
    )(flat_x, codebook)
    return out.reshape(nb * block_n)


def _sc_info():
    try:
        info = plsc.get_sparse_core_info()
        return info.num_cores, info.num_subcores
    except Exception:
        return 2, 16


def _make_sc_gather(n_part, d, part_off, chunk):
    nc, ns = _sc_info()
    nw = nc * ns
    b_per_w = n_part // nw
    assert b_per_w % chunk == 0 and chunk % 8 == 0
    n_chunks = b_per_w // chunk
    mesh = plsc.VectorSubcoreMesh(core_axis_name="c", subcore_axis_name="s")

    @functools.partial(
        pl.kernel,
        mesh=mesh,
        scratch_types=[
            pltpu.VMEM((b_per_w,), jnp.int32),
            pltpu.VMEM((chunk, d), jnp.float32),
            pltpu.VMEM((chunk, d), jnp.float32),
            pltpu.SemaphoreType.DMA,
            pltpu.SemaphoreType.DMA,
            pltpu.SemaphoreType.DMA,
            pltpu.SemaphoreType.DMA,
        ],
    )
    def gather_kernel(table_hbm, idx_hbm, out_hbm,
                      idx_v, rows0, rows1, gsem0, gsem1, ssem0, ssem1):
        wid = lax.axis_index("s") * nc + lax.axis_index("c")
        base = wid * b_per_w
        rows = (rows0, rows1)
        gsem = (gsem0, gsem1)
        ssem = (ssem0, ssem1)
        pltpu.sync_copy(idx_hbm.at[pl.ds(base, b_per_w)], idx_v)
        gaths = [
            pltpu.async_copy(
                table_hbm.at[idx_v.at[pl.ds(c * chunk, chunk)]],
                rows[c], gsem[c])
            for c in range(n_chunks)
        ]
        stores = []
        for c in range(n_chunks):
            gaths[c].wait()
            stores.append(pltpu.async_copy(
                rows[c],
                out_hbm.at[pl.ds(part_off + base + c * chunk, chunk)],
                ssem[c]))
        for st in stores:
            st.wait()

    return gather_kernel


def _alloc_body(o_ref):
    o_ref[...] = jnp.zeros_like(o_ref)


def _alloc_uninit(n, d):
    return pl.pallas_call(
        _alloc_body,
        grid=(1,),
        out_specs=pl.BlockSpec((8, 128), lambda i: (0, 0)),
        out_shape=jax.ShapeDtypeStruct((n, d), jnp.float32),
    )()


def kernel(x, codebook):
    d = x.shape[-1]
    flat_x = x.reshape(-1, d)
    n = flat_x.shape[0]
    parts = 2
    block_n = 2048
    np_ = n // parts
    nb = np_ // block_n
    out_ref = jax.new_ref(_alloc_uninit(n, d))
    for p in range(parts):
        idx = _tc_argmin(flat_x, codebook, block_n, p * nb, nb)
        _make_sc_gather(np_, d, p * np_, chunk=128)(codebook, idx, out_ref)
    return out_ref[...].reshape(x.shape)
